# baseline (speedup 1.0000x reference)
import functools

import jax
import jax.numpy as jnp
from jax import lax
from jax.experimental import pallas as pl
from jax.experimental.pallas import tpu as pltpu
from jax.experimental.pallas import tpu_sc as plsc

VOCAB = 100000
D = 128
L = 16384

NC = 2
NS = 16
NW = NC * NS
IDS_PER_W = L // NW
CHUNK = 128
NCHUNK = IDS_PER_W // CHUNK
NLANE = 16
NVREG = D // NLANE

_sc_mesh = plsc.VectorSubcoreMesh(core_axis_name="c", subcore_axis_name="s")


@functools.partial(
    pl.kernel,
    mesh=_sc_mesh,
    out_type=jax.ShapeDtypeStruct((NW, D), jnp.float32),
    scratch_types=[
        pltpu.VMEM((NCHUNK, CHUNK), jnp.int32),
        pltpu.VMEM((CHUNK, D), jnp.float32),
        pltpu.VMEM((D,), jnp.float32),
        pltpu.SemaphoreType.DMA,
    ],
)
def _gather_sum(ids_hbm, emb_hbm, out_hbm, idx_v, rows_v, out_v, sem):
    wid = lax.axis_index("s") * NC + lax.axis_index("c")
    pltpu.sync_copy(ids_hbm.at[wid], idx_v)
    acc = tuple(jnp.zeros((NLANE,), jnp.float32) for _ in range(NVREG))
    for k in range(NCHUNK):
        pltpu.async_copy(emb_hbm.at[idx_v.at[k]], rows_v, sem).wait()

        def body(i, carry):
            return tuple(
                carry[j] + rows_v[i, pl.ds(j * NLANE, NLANE)]
                for j in range(NVREG)
            )

        acc = lax.fori_loop(0, CHUNK, body, acc)
    for j in range(NVREG):
        out_v[pl.ds(j * NLANE, NLANE)] = acc[j]
    pltpu.sync_copy(out_v, out_hbm.at[wid])


BLK = 25088


def _matvec_body(p_ref, w_ref, b_ref, o_ref):
    s = jnp.sum(p_ref[...], axis=0, keepdims=True)
    o_ref[...] = (
        lax.dot_general(
            s, w_ref[...], (((1,), (1,)), ((), ())),
            preferred_element_type=jnp.float32,
        )
        + b_ref[...]
    )


def kernel(context_ids, embedding, W, b):
    ids3 = context_ids.reshape(NW, NCHUNK, CHUNK)
    partials = _gather_sum(ids3, embedding)
    out = pl.pallas_call(
        _matvec_body,
        grid=(pl.cdiv(VOCAB, BLK),),
        in_specs=[
            pl.BlockSpec((NW, D), lambda i: (0, 0)),
            pl.BlockSpec((BLK, D), lambda i: (i, 0)),
            pl.BlockSpec((1, BLK), lambda i: (0, i)),
        ],
        out_specs=pl.BlockSpec((1, BLK), lambda i: (0, i)),
        out_shape=jax.ShapeDtypeStruct((1, VOCAB), jnp.float32),
    )(partials, W, b.reshape(1, VOCAB))
    return out


# speedup vs baseline: 1.0415x; 1.0415x over previous
import functools

import jax
import jax.numpy as jnp
from jax import lax
from jax.experimental import pallas as pl
from jax.experimental.pallas import tpu as pltpu
from jax.experimental.pallas import tpu_sc as plsc

VOCAB = 100000
D = 128
L = 16384

NC = 2
NS = 16
NW = NC * NS
IDS_PER_W = L // NW
CHUNK = 128
NCHUNK = IDS_PER_W // CHUNK
NLANE = 16
NVREG = D // NLANE

_sc_mesh = plsc.VectorSubcoreMesh(core_axis_name="c", subcore_axis_name="s")


UNROLL = 4


@functools.partial(
    pl.kernel,
    mesh=_sc_mesh,
    out_type=jax.ShapeDtypeStruct((NW, D), jnp.float32),
    scratch_types=[
        pltpu.VMEM((NCHUNK, CHUNK), jnp.int32),
        pltpu.VMEM((2, CHUNK, D), jnp.float32),
        pltpu.VMEM((D,), jnp.float32),
        pltpu.SemaphoreType.DMA,
        pltpu.SemaphoreType.DMA,
    ],
)
def _gather_sum(ids_hbm, emb_hbm, out_hbm, idx_v, rows_v, out_v, sem0, sem1):
    sems = (sem0, sem1)
    wid = lax.axis_index("s") * NC + lax.axis_index("c")
    pltpu.sync_copy(ids_hbm.at[wid], idx_v)
    copies = [pltpu.async_copy(emb_hbm.at[idx_v.at[0]], rows_v.at[0], sems[0])]
    acc = tuple(jnp.zeros((NLANE,), jnp.float32) for _ in range(NVREG))
    for k in range(NCHUNK):
        if k + 1 < NCHUNK:
            copies.append(
                pltpu.async_copy(
                    emb_hbm.at[idx_v.at[k + 1]], rows_v.at[(k + 1) % 2],
                    sems[(k + 1) % 2],
                )
            )
        copies[k].wait()
        buf = rows_v.at[k % 2]

        def body(i, carry):
            for u in range(UNROLL):
                carry = tuple(
                    carry[j] + buf[i * UNROLL + u, pl.ds(j * NLANE, NLANE)]
                    for j in range(NVREG)
                )
            return carry

        acc = lax.fori_loop(0, CHUNK // UNROLL, body, acc)
    for j in range(NVREG):
        out_v[pl.ds(j * NLANE, NLANE)] = acc[j]
    pltpu.sync_copy(out_v, out_hbm.at[wid])


BLK = 25088


def _matvec_body(p_ref, w_ref, b_ref, o_ref):
    s = jnp.sum(p_ref[...], axis=0, keepdims=True)
    o_ref[...] = (
        lax.dot_general(
            s, w_ref[...], (((1,), (1,)), ((), ())),
            preferred_element_type=jnp.float32,
        )
        + b_ref[...]
    )


def kernel(context_ids, embedding, W, b):
    ids3 = context_ids.reshape(NW, NCHUNK, CHUNK)
    partials = _gather_sum(ids3, embedding)
    out = pl.pallas_call(
        _matvec_body,
        grid=(pl.cdiv(VOCAB, BLK),),
        in_specs=[
            pl.BlockSpec((NW, D), lambda i: (0, 0)),
            pl.BlockSpec((BLK, D), lambda i: (i, 0)),
            pl.BlockSpec((1, BLK), lambda i: (0, i)),
        ],
        out_specs=pl.BlockSpec((1, BLK), lambda i: (0, i)),
        out_shape=jax.ShapeDtypeStruct((1, VOCAB), jnp.float32),
    )(partials, W, b.reshape(1, VOCAB))
    return out
